# baseline (speedup 1.0000x reference)
import functools

import jax
import jax.numpy as jnp
from jax import lax
from jax.experimental import pallas as pl
from jax.experimental.pallas import tpu as pltpu
from jax.experimental.pallas import tpu_sc as plsc

_NUM_CORES = 2
_NUM_SUBCORES = 16
_NUM_WORKERS = _NUM_CORES * _NUM_SUBCORES
_LANES = 16
_STREAM_ROWS = 208


@functools.lru_cache(maxsize=None)
def _build(B, V, D):
    b_per_w = B // _NUM_WORKERS
    s_rows = _STREAM_ROWS
    d_rows = b_per_w - s_rows
    mesh = plsc.VectorSubcoreMesh(core_axis_name="c", subcore_axis_name="s")

    @functools.partial(
        pl.kernel,
        mesh=mesh,
        out_type=jax.ShapeDtypeStruct((B, D), jnp.float32),
        compiler_params=pltpu.CompilerParams(use_tc_tiling_on_sc=False),
        scratch_types=[
            pltpu.VMEM((b_per_w,), jnp.int32),
            pltpu.VMEM((b_per_w, D), jnp.float32),
            pltpu.SemaphoreType.DMA,
            pltpu.SemaphoreType.DMA,
        ],
    )
    def k(table_hbm, idx_hbm, out_hbm, idx_v, rows_v, ssem, dsem):
        wid = lax.axis_index("s") * _NUM_CORES + lax.axis_index("c")
        base = wid * b_per_w

        pltpu.sync_copy(idx_hbm.at[pl.ds(base, b_per_w)], idx_v)

        pltpu.async_copy(
            table_hbm.at[idx_v.at[pl.ds(0, s_rows)]],
            rows_v.at[pl.ds(0, s_rows)],
            ssem,
        )

        def grp(g, _):
            r = s_rows + g * _LANES
            v = idx_v[pl.ds(r, _LANES)]
            for j in range(_LANES):
                pltpu.async_copy(table_hbm.at[v[j]], rows_v.at[r + j], dsem)
            return ()

        lax.fori_loop(0, d_rows // _LANES, grp, (), unroll=2)

        pltpu.make_async_copy(
            table_hbm.at[pl.ds(0, d_rows)],
            rows_v.at[pl.ds(s_rows, d_rows)],
            dsem,
        ).wait()
        pltpu.make_async_copy(
            table_hbm.at[pl.ds(0, s_rows)],
            rows_v.at[pl.ds(0, s_rows)],
            ssem,
        ).wait()

        pltpu.sync_copy(rows_v, out_hbm.at[pl.ds(base, b_per_w)])

    return k


def kernel(genre_id, embedding_table):
    if genre_id.ndim == 2 and genre_id.shape[1] == 1:
        genre_id = genre_id.squeeze(1)
    B = genre_id.shape[0]
    V, D = embedding_table.shape
    idx = genre_id.astype(jnp.int32)
    return _build(B, V, D)(embedding_table, idx)


# speedup vs baseline: 1.0113x; 1.0113x over previous
import functools

import jax
import jax.numpy as jnp
from jax import lax
from jax.experimental import pallas as pl
from jax.experimental.pallas import tpu as pltpu
from jax.experimental.pallas import tpu_sc as plsc

_NUM_CORES = 2
_NUM_SUBCORES = 16
_NUM_WORKERS = _NUM_CORES * _NUM_SUBCORES
_LANES = 16


@functools.lru_cache(maxsize=None)
def _build(B, V, D):
    b_per_w = B // _NUM_WORKERS
    n_groups = b_per_w // _LANES
    mesh = plsc.VectorSubcoreMesh(core_axis_name="c", subcore_axis_name="s")

    @functools.partial(
        pl.kernel,
        mesh=mesh,
        out_type=jax.ShapeDtypeStruct((B, D), jnp.float32),
        compiler_params=pltpu.CompilerParams(use_tc_tiling_on_sc=False),
        scratch_types=[
            pltpu.VMEM((b_per_w,), jnp.int32),
            pltpu.VMEM((b_per_w, D), jnp.float32),
            pltpu.SemaphoreType.DMA,
        ],
    )
    def k(table_hbm, idx_hbm, out_hbm, idx_v, rows_v, sem):
        wid = lax.axis_index("s") * _NUM_CORES + lax.axis_index("c")
        base = wid * b_per_w

        pltpu.sync_copy(idx_hbm.at[pl.ds(base, b_per_w)], idx_v)

        def gather_group(g, _):
            v = idx_v[pl.ds(g * _LANES, _LANES)]
            for j in range(_LANES):
                pltpu.async_copy(
                    table_hbm.at[v[j]], rows_v.at[g * _LANES + j], sem
                )
            return ()

        lax.fori_loop(0, n_groups, gather_group, (), unroll=2)

        pltpu.make_async_copy(
            table_hbm.at[pl.ds(0, b_per_w)], rows_v, sem
        ).wait()

        pltpu.sync_copy(rows_v, out_hbm.at[pl.ds(base, b_per_w)])

    return k


def kernel(genre_id, embedding_table):
    if genre_id.ndim == 2 and genre_id.shape[1] == 1:
        genre_id = genre_id.squeeze(1)
    B = genre_id.shape[0]
    V, D = embedding_table.shape
    idx = genre_id.astype(jnp.int32)
    return _build(B, V, D)(embedding_table, idx)


# speedup vs baseline: 1.5177x; 1.5007x over previous
import functools

import jax
import jax.numpy as jnp
from jax import lax
from jax.experimental import pallas as pl
from jax.experimental.pallas import tpu as pltpu
from jax.experimental.pallas import tpu_sc as plsc

_NUM_CORES = 2
_NUM_SUBCORES = 16
_NUM_WORKERS = _NUM_CORES * _NUM_SUBCORES
_LANES = 16


@functools.lru_cache(maxsize=None)
def _build(B, V, D):
    b_per_w = B // _NUM_WORKERS
    n_groups = b_per_w // _LANES
    mesh = plsc.VectorSubcoreMesh(core_axis_name="c", subcore_axis_name="s")

    @functools.partial(
        pl.kernel,
        mesh=mesh,
        out_type=jax.ShapeDtypeStruct((B, D), jnp.float32),
        scratch_types=[
            pltpu.VMEM((b_per_w,), jnp.int32),
            pltpu.VMEM((b_per_w, D), jnp.float32),
            pltpu.SemaphoreType.DMA,
        ],
    )
    def k(table_hbm, idx_hbm, out_hbm, idx_v, rows_v, sem):
        wid = lax.axis_index("s") * _NUM_CORES + lax.axis_index("c")
        base = wid * b_per_w

        pltpu.sync_copy(idx_hbm.at[pl.ds(base, b_per_w)], idx_v)

        def gather_group(g, _):
            v = idx_v[pl.ds(g * _LANES, _LANES)]
            for j in range(_LANES):
                pltpu.async_copy(
                    table_hbm.at[v[j]], rows_v.at[g * _LANES + j], sem
                )
            return ()

        lax.fori_loop(0, n_groups, gather_group, (), unroll=2)

        pltpu.make_async_copy(
            table_hbm.at[pl.ds(0, b_per_w)], rows_v, sem
        ).wait()

        pltpu.sync_copy(rows_v, out_hbm.at[pl.ds(base, b_per_w)])

    return k


def kernel(genre_id, embedding_table):
    if genre_id.ndim == 2 and genre_id.shape[1] == 1:
        genre_id = genre_id.squeeze(1)
    B = genre_id.shape[0]
    V, D = embedding_table.shape
    idx = genre_id.astype(jnp.int32)
    return _build(B, V, D)(embedding_table, idx)
